# baseline (speedup 1.0000x reference)
import functools

import jax
import jax.numpy as jnp
import numpy as np
from jax.experimental import pallas as pl
from jax.experimental.pallas import tpu as pltpu


def _perm_matrix(width, f):
    wq = width // f
    k = np.arange(width)
    src = (k % wq) * f + (k // wq)
    m = np.zeros((width, width), np.float32)
    m[src, k] = 1.0
    return m


def _schedule(n):
    head = [4, 4, 8, 16]
    tail = [16, 8, 4, 4]
    mid_rows = n - sum(head) - sum(tail)
    assert mid_rows >= 0 and mid_rows % 32 == 0, n
    sizes = head + [32] * (mid_rows // 32) + tail
    sched = []
    off = 0
    for s in sizes:
        sched.append((off, s))
        off += s
    return sched


def _body(x_hbm, p_ref, o_hbm, xbuf, obuf, isem, osem, *, sched, W):
    Wo = W // 2
    P = p_ref[...]
    nb = len(sched)

    def rd(i):
        off, sz = sched[i]
        return pltpu.make_async_copy(
            x_hbm.at[pl.ds(off, sz)], xbuf.at[i % 2, pl.ds(0, sz)], isem.at[i])

    def wr(i):
        off, sz = sched[i]
        return pltpu.make_async_copy(
            obuf.at[i % 2, pl.ds(0, sz)], o_hbm.at[pl.ds(off, sz)], osem.at[i])

    rd(0).start()
    rd(1).start()
    for i in range(nb):
        off, sz = sched[i]
        slot = i % 2
        rd(i).wait()
        if i >= 2:
            wr(i - 2).wait()
        xb = xbuf[slot, :sz].astype(jnp.bfloat16)
        hblk = xb.shape[1]
        for fh in range(2):
            rows = xb[:, :, fh * W:(fh + 1) * W].reshape(sz * hblk, W)
            perm = jnp.dot(rows, P, preferred_element_type=jnp.float32)
            perm = perm.reshape(sz, hblk, W)
            obuf[slot, :sz, 2 * fh] = perm[:, :, :Wo]
            obuf[slot, :sz, 2 * fh + 1] = perm[:, :, Wo:]
        if i + 2 < nb:
            rd(i + 2).start()
        wr(i).start()
    wr(nb - 2).wait()
    wr(nb - 1).wait()


def kernel(x):
    B, C, H, W = x.shape
    f = 2
    Ho, Wo = H // f, W // f
    N = B * C
    xv = x.reshape(N, Ho, f * W)
    P = jnp.asarray(_perm_matrix(W, f), jnp.bfloat16)
    sched = _schedule(N)
    nb = len(sched)

    out = pl.pallas_call(
        functools.partial(_body, sched=sched, W=W),
        out_shape=jax.ShapeDtypeStruct((N, f * f, Ho, Wo), x.dtype),
        in_specs=[
            pl.BlockSpec(memory_space=pl.ANY),
            pl.BlockSpec(memory_space=pltpu.MemorySpace.VMEM),
        ],
        out_specs=pl.BlockSpec(memory_space=pl.ANY),
        scratch_shapes=[
            pltpu.VMEM((2, 32, Ho, f * W), jnp.float32),
            pltpu.VMEM((2, 32, f * f, Ho, Wo), jnp.float32),
            pltpu.SemaphoreType.DMA((nb,)),
            pltpu.SemaphoreType.DMA((nb,)),
        ],
        compiler_params=pltpu.CompilerParams(
            vmem_limit_bytes=58 * 2**20),
        cost_estimate=pl.CostEstimate(
            flops=N * H * W * W, transcendentals=0,
            bytes_accessed=2 * x.size * x.dtype.itemsize),
    )(xv, P)
    return out.reshape(B, C * f * f, Ho, Wo)


# speedup vs baseline: 1.0127x; 1.0127x over previous
import functools

import jax
import jax.numpy as jnp
import numpy as np
from jax.experimental import pallas as pl
from jax.experimental.pallas import tpu as pltpu


def _perm_matrix(width, f):
    wq = width // f
    k = np.arange(width)
    src = (k % wq) * f + (k // wq)
    m = np.zeros((width, width), np.float32)
    m[src, k] = 1.0
    return m


def _squeeze_body(x_ref, p_ref, o_ref):
    rblk, hblk, fw_total = x_ref.shape
    W = fw_total // 2
    Wo = W // 2
    P = p_ref[...]
    xb = x_ref[...].astype(jnp.bfloat16)
    for fh in range(2):
        rows = xb[:, :, fh * W:(fh + 1) * W].reshape(rblk * hblk, W)
        perm = jnp.dot(rows, P, preferred_element_type=jnp.float32)
        perm = perm.reshape(rblk, hblk, W)
        for fw in range(2):
            o_ref[:, fh * 2 + fw, :, :] = perm[:, :, fw * Wo:(fw + 1) * Wo]


def kernel(x):
    B, C, H, W = x.shape
    f = 2
    Ho, Wo = H // f, W // f
    N = B * C
    xv = x.reshape(N, Ho, f * W)
    P = jnp.asarray(_perm_matrix(W, f), jnp.bfloat16)

    rblk = 32
    grid = (N // rblk,)

    out = pl.pallas_call(
        _squeeze_body,
        out_shape=jax.ShapeDtypeStruct((N, f * f, Ho, Wo), x.dtype),
        grid=grid,
        in_specs=[
            pl.BlockSpec((rblk, Ho, f * W), lambda g: (g, 0, 0)),
            pl.BlockSpec((W, W), lambda g: (0, 0)),
        ],
        out_specs=pl.BlockSpec((rblk, f * f, Ho, Wo), lambda g: (g, 0, 0, 0)),
        compiler_params=pltpu.CompilerParams(
            dimension_semantics=("parallel",),
            vmem_limit_bytes=58 * 2**20),
        cost_estimate=pl.CostEstimate(
            flops=N * H * W * W, transcendentals=0,
            bytes_accessed=2 * x.size * x.dtype.itemsize),
    )(xv, P)
    return out.reshape(B, C * f * f, Ho, Wo)


# speedup vs baseline: 1.0181x; 1.0053x over previous
import functools

import jax
import jax.numpy as jnp
import numpy as np
from jax.experimental import pallas as pl
from jax.experimental.pallas import tpu as pltpu


def _perm_matrix(width, f):
    wq = width // f
    k = np.arange(width)
    src = (k % wq) * f + (k // wq)
    m = np.zeros((width, width), np.float32)
    m[src, k] = 1.0
    return m


def _squeeze_body(x_ref, p_ref, o_ref):
    rblk, hblk, fw_total = x_ref.shape
    W = fw_total // 2
    Wo = W // 2
    P = p_ref[...]
    xb = x_ref[...].astype(jnp.bfloat16)
    for fh in range(2):
        rows = xb[:, :, fh * W:(fh + 1) * W].reshape(rblk * hblk, W)
        perm = jnp.dot(rows, P, preferred_element_type=jnp.float32)
        perm = perm.reshape(rblk, hblk, W)
        for fw in range(2):
            o_ref[:, fh * 2 + fw, :, :] = perm[:, :, fw * Wo:(fw + 1) * Wo]


def kernel(x):
    B, C, H, W = x.shape
    f = 2
    Ho, Wo = H // f, W // f
    N = B * C
    xv = x.reshape(N, Ho, f * W)
    P = jnp.asarray(_perm_matrix(W, f), jnp.bfloat16)

    rblk = 48
    grid = (N // rblk,)

    out = pl.pallas_call(
        _squeeze_body,
        out_shape=jax.ShapeDtypeStruct((N, f * f, Ho, Wo), x.dtype),
        grid=grid,
        in_specs=[
            pl.BlockSpec((rblk, Ho, f * W), lambda g: (g, 0, 0)),
            pl.BlockSpec((W, W), lambda g: (0, 0)),
        ],
        out_specs=pl.BlockSpec((rblk, f * f, Ho, Wo), lambda g: (g, 0, 0, 0)),
        compiler_params=pltpu.CompilerParams(
            dimension_semantics=("parallel",),
            vmem_limit_bytes=58 * 2**20),
        cost_estimate=pl.CostEstimate(
            flops=N * H * W * W, transcendentals=0,
            bytes_accessed=2 * x.size * x.dtype.itemsize),
    )(xv, P)
    return out.reshape(B, C * f * f, Ho, Wo)
